# baseline (speedup 1.0000x reference)
import jax
import jax.numpy as jnp
from jax.experimental import pallas as pl

_DIM = 4096
_EXPERTS = 64
_TOKENS = 8192
_TILE = 1024
_HD = _DIM // 2


def _gate_kernel(xa_ref, xb_ref, wta_ref, wtb_ref, b_ref, gs_ref, ts_ref, ti_ref):
    dn = (((1,), (0,)), ((), ()))
    la = jax.lax.dot_general(
        xa_ref[...], wta_ref[...], dn, preferred_element_type=jnp.float32
    )
    lb = jax.lax.dot_general(
        xb_ref[...], wtb_ref[...], dn, preferred_element_type=jnp.float32
    )
    logits = la + lb + b_ref[...]
    m = jnp.max(logits, axis=1, keepdims=True)
    e = jnp.exp(logits - m)
    s = jnp.sum(e, axis=1, keepdims=True)
    gate = e / s
    gs_ref[...] = gate

    idx = jax.lax.broadcasted_iota(jnp.int32, gate.shape, 1)
    m1 = jnp.max(gate, axis=1, keepdims=True)
    i1 = jnp.min(jnp.where(gate == m1, idx, _EXPERTS), axis=1, keepdims=True)
    masked = jnp.where(idx == i1, -jnp.inf, gate)
    m2 = jnp.max(masked, axis=1, keepdims=True)
    i2 = jnp.min(jnp.where(masked == m2, idx, _EXPERTS), axis=1, keepdims=True)
    ts_ref[...] = jnp.concatenate([m1, m2], axis=1)
    ti_ref[...] = jnp.concatenate([i1, i2], axis=1)


def kernel(x, W, b):
    wt = W.T
    b2 = b.reshape(1, _EXPERTS)
    grid = (_TOKENS // _TILE,)
    out_shape = (
        jax.ShapeDtypeStruct((_TOKENS, _EXPERTS), jnp.float32),
        jax.ShapeDtypeStruct((_TOKENS, 2), jnp.float32),
        jax.ShapeDtypeStruct((_TOKENS, 2), jnp.int32),
    )
    gs, ts, ti = pl.pallas_call(
        _gate_kernel,
        grid=grid,
        in_specs=[
            pl.BlockSpec((_TILE, _HD), lambda i: (i, 0)),
            pl.BlockSpec((_TILE, _HD), lambda i: (i, 1)),
            pl.BlockSpec((_HD, _EXPERTS), lambda i: (0, 0)),
            pl.BlockSpec((_HD, _EXPERTS), lambda i: (1, 0)),
            pl.BlockSpec((1, _EXPERTS), lambda i: (0, 0)),
        ],
        out_specs=[
            pl.BlockSpec((_TILE, _EXPERTS), lambda i: (i, 0)),
            pl.BlockSpec((_TILE, 2), lambda i: (i, 0)),
            pl.BlockSpec((_TILE, 2), lambda i: (i, 0)),
        ],
        out_shape=out_shape,
    )(x, x, wt, wt, b2)
    return (gs, ts, ti)
